# baseline (speedup 1.0000x reference)
import functools

import jax
import jax.numpy as jnp
from jax import lax
from jax.experimental import pallas as pl
from jax.experimental.pallas import tpu as pltpu
from jax.experimental.pallas import tpu_sc as plsc

W = 5
D = 64
B = 1024
L = 200
N = B * L


def _sc_gather(sup_flat, table128):
    info = plsc.get_sparse_core_info()
    nw = info.num_cores * info.num_subcores
    per_w = N // nw
    chunk = 800
    n_chunks = per_w // chunk

    mesh = plsc.VectorSubcoreMesh(core_axis_name="c", subcore_axis_name="s")

    @functools.partial(
        pl.kernel,
        out_type=jax.ShapeDtypeStruct((N, 2 * D), jnp.float32),
        mesh=mesh,
        scratch_types=[
            pltpu.VMEM((chunk,), jnp.int32),
            pltpu.VMEM((chunk, 2 * D), jnp.float32),
            pltpu.SemaphoreType.DMA,
        ],
    )
    def gather_kernel(table_hbm, idx_hbm, out_hbm, idx_v, rows_v, sem):
        wid = lax.axis_index("s") * info.num_cores + lax.axis_index("c")

        def body(i, carry):
            base = wid * per_w + i * chunk
            pltpu.sync_copy(idx_hbm.at[pl.ds(base, chunk)], idx_v)
            pltpu.async_copy(table_hbm.at[idx_v], rows_v, sem).wait()
            pltpu.sync_copy(rows_v, out_hbm.at[pl.ds(base, chunk)])
            return carry

        lax.fori_loop(0, n_chunks, body, 0)

    return gather_kernel(table128, sup_flat)


def _window_body(emb2_ref, inputs_ref, out_ref):
    e2 = emb2_ref[...]
    par = (inputs_ref[...] & 1) == 1
    e = jnp.where(par, e2[:, :, D:], e2[:, :, :D])
    bb = e.shape[0]
    parts = [e]
    for k in range(1, W):
        z = jnp.zeros((bb, k, D), jnp.float32)
        parts.append(jnp.concatenate([z, e[:, : L - k, :]], axis=1))
    out_ref[...] = jnp.concatenate(parts, axis=2)


def _tc_window(emb2, inputs):
    bb = 8
    return pl.pallas_call(
        _window_body,
        grid=(B // bb,),
        in_specs=[
            pl.BlockSpec((bb, L, 2 * D), lambda i: (i, 0, 0)),
            pl.BlockSpec((bb, L, 1), lambda i: (i, 0, 0)),
        ],
        out_specs=pl.BlockSpec((bb, L, W * D), lambda i: (i, 0, 0)),
        out_shape=jax.ShapeDtypeStruct((B, L, W * D), jnp.float32),
    )(emb2, inputs)


def kernel(inputs, table):
    idx_flat = inputs.reshape(-1).astype(jnp.int32)
    sup_flat = idx_flat >> 1
    table128 = table.reshape(500000, 2 * D)
    emb2 = _sc_gather(sup_flat, table128)
    return _tc_window(emb2.reshape(B, L, 2 * D), inputs.astype(jnp.int32).reshape(B, L, 1))
